# baseline (speedup 1.0000x reference)
import functools

import jax
import jax.numpy as jnp
from jax import lax
from jax.experimental import pallas as pl
from jax.experimental.pallas import tpu as pltpu
from jax.experimental.pallas import tpu_sc as plsc

N_TOKENS = 32768
ENCODING_DIM = 64
NUM_CODEWORDS = 1024

N_BLOCK = 2048


def _vq_tc_body(x_ref, cw_ref, idx_ref, dist_ref):
    cw = cw_ref[...]
    c2 = jnp.sum(cw * cw, axis=1, keepdims=True)
    x = x_ref[...]
    x2 = jnp.sum(x * x, axis=1)[None, :]
    ft = lax.dot_general(cw, -2.0 * x, (((1,), (1,)), ((), ())),
                         preferred_element_type=jnp.float32)
    ft = ft + c2
    minf = jnp.min(ft, axis=0, keepdims=True)
    rowf = lax.broadcasted_iota(
        jnp.int32, (NUM_CODEWORDS, 1), 0).astype(jnp.float32)
    idxf = jnp.min(jnp.where(ft == minf, rowf, float(NUM_CODEWORDS)),
                   axis=0, keepdims=True)
    idx_ref[...] = idxf.astype(jnp.int32)[None]
    dist_ref[...] = jnp.sqrt(jnp.maximum(minf + x2, 0.0))[None]


def _vq_distances(input_data, codewords):
    n = input_data.shape[0]
    nb = n // N_BLOCK
    idx3, dist3 = pl.pallas_call(
        _vq_tc_body,
        grid=(nb,),
        in_specs=[
            pl.BlockSpec((N_BLOCK, ENCODING_DIM), lambda i: (i, 0)),
            pl.BlockSpec((NUM_CODEWORDS, ENCODING_DIM), lambda i: (0, 0)),
        ],
        out_specs=[
            pl.BlockSpec((1, 1, N_BLOCK), lambda i: (i, 0, 0)),
            pl.BlockSpec((1, 1, N_BLOCK), lambda i: (i, 0, 0)),
        ],
        out_shape=[
            jax.ShapeDtypeStruct((nb, 1, N_BLOCK), jnp.int32),
            jax.ShapeDtypeStruct((nb, 1, N_BLOCK), jnp.float32),
        ],
        compiler_params=pltpu.CompilerParams(
            dimension_semantics=("parallel",)),
    )(input_data, codewords)
    return idx3.reshape(-1), dist3.reshape(-1)


GATHER_CHUNK = 128
NBUF = 4


def _make_sc_gather(b):
    info = plsc.get_sparse_core_info()
    nc, ns = info.num_cores, info.num_subcores
    nw = nc * ns
    b_per_w = b // nw
    n_chunks = b_per_w // GATHER_CHUNK
    mesh = plsc.VectorSubcoreMesh(core_axis_name="c", subcore_axis_name="s")

    @functools.partial(
        pl.kernel, mesh=mesh,
        out_type=jax.ShapeDtypeStruct((b, ENCODING_DIM), jnp.float32),
        compiler_params=pltpu.CompilerParams(use_tc_tiling_on_sc=False),
        scratch_types=[
            pltpu.VMEM((n_chunks, GATHER_CHUNK), jnp.int32),
            pltpu.VMEM((NBUF, GATHER_CHUNK, ENCODING_DIM), jnp.float32),
        ] + [pltpu.SemaphoreType.DMA] * (2 * NBUF),
    )
    def gather(table_hbm, idx_hbm, out_hbm, idx_v, rows_v, *sems):
        wid = lax.axis_index("s") * nc + lax.axis_index("c")
        base = wid * b_per_w
        gsem = sems[:NBUF]
        wsem = sems[NBUF:]
        pltpu.sync_copy(idx_hbm.at[wid], idx_v)

        def start_gather(c):
            return pltpu.async_copy(
                table_hbm.at[idx_v.at[c]], rows_v.at[c % NBUF], gsem[c % NBUF])

        def start_wb(c):
            return pltpu.async_copy(
                rows_v.at[c % NBUF],
                out_hbm.at[pl.ds(base + c * GATHER_CHUNK, GATHER_CHUNK)],
                wsem[c % NBUF])

        gathers = [None] * NBUF
        wbs = [None] * NBUF
        for c in range(min(NBUF, n_chunks)):
            gathers[c] = start_gather(c)
        for c in range(n_chunks):
            buf = c % NBUF
            nxt = c + NBUF
            gathers[buf].wait()
            wbs[buf] = start_wb(c)
            if nxt < n_chunks:
                wbs[buf].wait()
                gathers[buf] = start_gather(nxt)
        for c in range(max(0, n_chunks - NBUF), n_chunks):
            wbs[c % NBUF].wait()

    return gather, nw, n_chunks


def kernel(input_data, codewords):
    x = input_data.reshape(-1, codewords.shape[1])
    quantized_indices, quantized_distances = _vq_distances(x, codewords)
    gather, nw, n_chunks = _make_sc_gather(x.shape[0])
    idx3 = quantized_indices.reshape(nw, n_chunks, GATHER_CHUNK)
    quantized_data = gather(codewords, idx3)
    return (quantized_indices, quantized_distances, quantized_data)


# speedup vs baseline: 1.0140x; 1.0140x over previous
import functools

import jax
import jax.numpy as jnp
from jax import lax
from jax.experimental import pallas as pl
from jax.experimental.pallas import tpu as pltpu
from jax.experimental.pallas import tpu_sc as plsc

N_TOKENS = 32768
ENCODING_DIM = 64
NUM_CODEWORDS = 1024

N_BLOCK = 4096


def _vq_tc_body(x_ref, cw_ref, idx_ref, dist_ref):
    cw = cw_ref[...]
    c2 = jnp.sum(cw * cw, axis=1, keepdims=True)
    x = x_ref[...]
    x2 = jnp.sum(x * x, axis=1)[None, :]
    ft = lax.dot_general(cw, -2.0 * x, (((1,), (1,)), ((), ())),
                         preferred_element_type=jnp.float32)
    ft = ft + c2
    minf = jnp.min(ft, axis=0, keepdims=True)
    rowf = lax.broadcasted_iota(
        jnp.int32, (NUM_CODEWORDS, 1), 0).astype(jnp.float32)
    idxf = jnp.min(jnp.where(ft == minf, rowf, float(NUM_CODEWORDS)),
                   axis=0, keepdims=True)
    idx_ref[...] = idxf.astype(jnp.int32)[None]
    dist_ref[...] = jnp.sqrt(jnp.maximum(minf + x2, 0.0))[None]


def _vq_distances(input_data, codewords):
    n = input_data.shape[0]
    nb = n // N_BLOCK
    idx3, dist3 = pl.pallas_call(
        _vq_tc_body,
        grid=(nb,),
        in_specs=[
            pl.BlockSpec((N_BLOCK, ENCODING_DIM), lambda i: (i, 0)),
            pl.BlockSpec((NUM_CODEWORDS, ENCODING_DIM), lambda i: (0, 0)),
        ],
        out_specs=[
            pl.BlockSpec((1, 1, N_BLOCK), lambda i: (i, 0, 0)),
            pl.BlockSpec((1, 1, N_BLOCK), lambda i: (i, 0, 0)),
        ],
        out_shape=[
            jax.ShapeDtypeStruct((nb, 1, N_BLOCK), jnp.int32),
            jax.ShapeDtypeStruct((nb, 1, N_BLOCK), jnp.float32),
        ],
        compiler_params=pltpu.CompilerParams(
            dimension_semantics=("parallel",)),
    )(input_data, codewords)
    return idx3.reshape(-1), dist3.reshape(-1)


GATHER_CHUNK = 128
NBUF = 8


def _make_sc_gather(b):
    info = plsc.get_sparse_core_info()
    nc, ns = info.num_cores, info.num_subcores
    nw = nc * ns
    b_per_w = b // nw
    n_chunks = b_per_w // GATHER_CHUNK
    mesh = plsc.VectorSubcoreMesh(core_axis_name="c", subcore_axis_name="s")

    @functools.partial(
        pl.kernel, mesh=mesh,
        out_type=jax.ShapeDtypeStruct((b, ENCODING_DIM), jnp.float32),
        compiler_params=pltpu.CompilerParams(use_tc_tiling_on_sc=False),
        scratch_types=[
            pltpu.VMEM((n_chunks, GATHER_CHUNK), jnp.int32),
            pltpu.VMEM((NBUF, GATHER_CHUNK, ENCODING_DIM), jnp.float32),
        ] + [pltpu.SemaphoreType.DMA] * (2 * NBUF),
    )
    def gather(table_hbm, idx_hbm, out_hbm, idx_v, rows_v, *sems):
        wid = lax.axis_index("s") * nc + lax.axis_index("c")
        base = wid * b_per_w
        gsem = sems[:NBUF]
        wsem = sems[NBUF:]
        pltpu.sync_copy(idx_hbm.at[wid], idx_v)

        def start_gather(c):
            return pltpu.async_copy(
                table_hbm.at[idx_v.at[c]], rows_v.at[c % NBUF], gsem[c % NBUF])

        def start_wb(c):
            return pltpu.async_copy(
                rows_v.at[c % NBUF],
                out_hbm.at[pl.ds(base + c * GATHER_CHUNK, GATHER_CHUNK)],
                wsem[c % NBUF])

        gathers = [None] * NBUF
        wbs = [None] * NBUF
        for c in range(min(NBUF, n_chunks)):
            gathers[c] = start_gather(c)
        for c in range(n_chunks):
            buf = c % NBUF
            nxt = c + NBUF
            gathers[buf].wait()
            wbs[buf] = start_wb(c)
            if nxt < n_chunks:
                wbs[buf].wait()
                gathers[buf] = start_gather(nxt)
        for c in range(max(0, n_chunks - NBUF), n_chunks):
            wbs[c % NBUF].wait()

    return gather, nw, n_chunks


def kernel(input_data, codewords):
    x = input_data.reshape(-1, codewords.shape[1])
    quantized_indices, quantized_distances = _vq_distances(x, codewords)
    gather, nw, n_chunks = _make_sc_gather(x.shape[0])
    idx3 = quantized_indices.reshape(nw, n_chunks, GATHER_CHUNK)
    quantized_data = gather(codewords, idx3)
    return (quantized_indices, quantized_distances, quantized_data)


# speedup vs baseline: 1.0163x; 1.0022x over previous
import functools

import jax
import jax.numpy as jnp
from jax import lax
from jax.experimental import pallas as pl
from jax.experimental.pallas import tpu as pltpu
from jax.experimental.pallas import tpu_sc as plsc

N_TOKENS = 32768
ENCODING_DIM = 64
NUM_CODEWORDS = 1024

N_BLOCK = 4096


def _vq_tc_body(x_ref, cw_ref, idx_ref, dist_ref):
    cw = cw_ref[...]
    c2 = jnp.sum(cw * cw, axis=1, keepdims=True)
    x = x_ref[...]
    x2 = jnp.sum(x * x, axis=1)[None, :]
    ft = lax.dot_general(cw, -2.0 * x, (((1,), (1,)), ((), ())),
                         preferred_element_type=jnp.float32)
    ft = ft + c2
    minf = jnp.min(ft, axis=0, keepdims=True)
    rowf = lax.broadcasted_iota(
        jnp.int32, (NUM_CODEWORDS, 1), 0).astype(jnp.float32)
    idxf = jnp.min(jnp.where(ft == minf, rowf, float(NUM_CODEWORDS)),
                   axis=0, keepdims=True)
    idx_ref[...] = idxf.astype(jnp.int32).reshape(N_BLOCK // 128, 128)
    dist_ref[...] = jnp.sqrt(jnp.maximum(minf + x2, 0.0)
                             ).reshape(N_BLOCK // 128, 128)


def _vq_distances(input_data, codewords):
    n = input_data.shape[0]
    nb = n // N_BLOCK
    idx3, dist3 = pl.pallas_call(
        _vq_tc_body,
        grid=(nb,),
        in_specs=[
            pl.BlockSpec((N_BLOCK, ENCODING_DIM), lambda i: (i, 0)),
            pl.BlockSpec((NUM_CODEWORDS, ENCODING_DIM), lambda i: (0, 0)),
        ],
        out_specs=[
            pl.BlockSpec((N_BLOCK // 128, 128), lambda i: (i, 0)),
            pl.BlockSpec((N_BLOCK // 128, 128), lambda i: (i, 0)),
        ],
        out_shape=[
            jax.ShapeDtypeStruct((n // 128, 128), jnp.int32),
            jax.ShapeDtypeStruct((n // 128, 128), jnp.float32),
        ],
        compiler_params=pltpu.CompilerParams(
            dimension_semantics=("parallel",)),
    )(input_data, codewords)
    return idx3.reshape(-1), dist3.reshape(-1)


GATHER_CHUNK = 128
NBUF = 8


def _make_sc_gather(b):
    info = plsc.get_sparse_core_info()
    nc, ns = info.num_cores, info.num_subcores
    nw = nc * ns
    b_per_w = b // nw
    n_chunks = b_per_w // GATHER_CHUNK
    mesh = plsc.VectorSubcoreMesh(core_axis_name="c", subcore_axis_name="s")

    @functools.partial(
        pl.kernel, mesh=mesh,
        out_type=jax.ShapeDtypeStruct((b, ENCODING_DIM), jnp.float32),
        compiler_params=pltpu.CompilerParams(use_tc_tiling_on_sc=False),
        scratch_types=[
            pltpu.VMEM((n_chunks, GATHER_CHUNK), jnp.int32),
            pltpu.VMEM((NBUF, GATHER_CHUNK, ENCODING_DIM), jnp.float32),
        ] + [pltpu.SemaphoreType.DMA] * (2 * NBUF),
    )
    def gather(table_hbm, idx_hbm, out_hbm, idx_v, rows_v, *sems):
        wid = lax.axis_index("s") * nc + lax.axis_index("c")
        base = wid * b_per_w
        gsem = sems[:NBUF]
        wsem = sems[NBUF:]
        pltpu.sync_copy(idx_hbm.at[wid], idx_v)

        def start_gather(c):
            return pltpu.async_copy(
                table_hbm.at[idx_v.at[c]], rows_v.at[c % NBUF], gsem[c % NBUF])

        def start_wb(c):
            return pltpu.async_copy(
                rows_v.at[c % NBUF],
                out_hbm.at[pl.ds(base + c * GATHER_CHUNK, GATHER_CHUNK)],
                wsem[c % NBUF])

        gathers = [None] * NBUF
        wbs = [None] * NBUF
        for c in range(min(NBUF, n_chunks)):
            gathers[c] = start_gather(c)
        for c in range(n_chunks):
            buf = c % NBUF
            nxt = c + NBUF
            gathers[buf].wait()
            wbs[buf] = start_wb(c)
            if nxt < n_chunks:
                wbs[buf].wait()
                gathers[buf] = start_gather(nxt)
        for c in range(max(0, n_chunks - NBUF), n_chunks):
            wbs[c % NBUF].wait()

    return gather, nw, n_chunks


def kernel(input_data, codewords):
    x = input_data.reshape(-1, codewords.shape[1])
    quantized_indices, quantized_distances = _vq_distances(x, codewords)
    gather, nw, n_chunks = _make_sc_gather(x.shape[0])
    idx3 = quantized_indices.reshape(nw, n_chunks, GATHER_CHUNK)
    quantized_data = gather(codewords, idx3)
    return (quantized_indices, quantized_distances, quantized_data)
